# baseline (speedup 1.0000x reference)
import jax
import jax.numpy as jnp
from jax import lax
from jax.experimental import pallas as pl
from jax.experimental.pallas import tpu as pltpu

B, SEQ, T, D = 1024, 50, 26, 32
SB = 5


def _tc_body(xt_ref, t0_ref, t1_ref, o_ref):
    tt = lax.broadcasted_iota(jnp.int32, (T, B), 0)
    t0 = t0_ref[...]
    t1 = t1_ref[...]
    for j in range(SB):
        mask = xt_ref[j, 0, :][None, :] == tt
        o_ref[j] = jnp.where(mask[:, None, :], t1[None], t0[None])


@jax.jit
def _run_tc(xt, t0b, t1b):
    return pl.pallas_call(
        _tc_body,
        out_shape=jax.ShapeDtypeStruct((SEQ, T, D, B), jnp.float32),
        grid=(SEQ // SB,),
        in_specs=[
            pl.BlockSpec((SB, 1, B), lambda s: (s, 0, 0)),
            pl.BlockSpec((D, B), lambda s: (0, 0)),
            pl.BlockSpec((D, B), lambda s: (0, 0)),
        ],
        out_specs=pl.BlockSpec((SB, T, D, B), lambda s: (s, 0, 0, 0)),
    )(xt, t0b, t1b)


def kernel(x, table):
    xt = x.T.reshape(SEQ, 1, B)
    t0b = jnp.broadcast_to(table[0][:, None], (D, B))
    t1b = jnp.broadcast_to(table[1][:, None], (D, B))
    o = _run_tc(xt, t0b, t1b)
    return o.transpose(3, 0, 1, 2)
